# baseline (speedup 1.0000x reference)
import functools

import jax
import jax.numpy as jnp
from jax import lax
from jax.experimental import pallas as pl
from jax.experimental.pallas import tpu as pltpu
from jax.experimental.pallas import tpu_sc as plsc

N = 16384
D = 2048
NEXP = 16
BLK = 512

NWORK = 32
RPW = N // NWORK
LANES = 16


def _dense_block(h_ref, ww_ref, wn_ref, bw_ref, bn_ref, noise_ref, full_ref):
    h = h_ref[...]
    logits = jax.lax.dot_general(
        h, ww_ref[...], (((1,), (1,)), ((), ())),
        preferred_element_type=jnp.float32) + bw_ref[...]
    nlin = jax.lax.dot_general(
        h, wn_ref[...], (((1,), (1,)), ((), ())),
        preferred_element_type=jnp.float32) + bn_ref[...]
    noisy = logits + noise_ref[...] * jax.nn.softplus(nlin)

    m = jnp.max(noisy, axis=1, keepdims=True)
    e = jnp.exp(noisy - m)
    full_ref[...] = e / jnp.sum(e, axis=1, keepdims=True)


def _dense_full(h, W_w, b_w, W_n, b_n, noise):
    grid = (N // BLK,)
    return pl.pallas_call(
        _dense_block,
        grid=grid,
        in_specs=[
            pl.BlockSpec((BLK, D), lambda i: (i, 0)),
            pl.BlockSpec((NEXP, D), lambda i: (0, 0)),
            pl.BlockSpec((NEXP, D), lambda i: (0, 0)),
            pl.BlockSpec((1, NEXP), lambda i: (0, 0)),
            pl.BlockSpec((1, NEXP), lambda i: (0, 0)),
            pl.BlockSpec((BLK, NEXP), lambda i: (i, 0)),
        ],
        out_specs=pl.BlockSpec((BLK, NEXP), lambda i: (i, 0)),
        out_shape=jax.ShapeDtypeStruct((N, NEXP), jnp.float32),
    )(h, W_w, W_n, b_w.reshape(1, NEXP), b_n.reshape(1, NEXP), noise)


@functools.partial(
    pl.kernel,
    mesh=plsc.VectorSubcoreMesh(core_axis_name="c", subcore_axis_name="s"),
    out_type=[
        jax.ShapeDtypeStruct((N * NEXP,), jnp.float32),
        jax.ShapeDtypeStruct((N * 2,), jnp.int32),
    ],
    scratch_types=[
        pltpu.VMEM((RPW * NEXP,), jnp.float32),
        pltpu.VMEM((RPW * NEXP,), jnp.float32),
        pltpu.VMEM((RPW * 2,), jnp.int32),
    ],
    compiler_params=pltpu.CompilerParams(needs_layout_passes=False),
)
def _sc_route(full_hbm, sparse_hbm, ix_hbm, fl_v, sp_v, ix_v):
    wid = lax.axis_index("s") * 2 + lax.axis_index("c")
    base = wid * RPW
    pltpu.sync_copy(full_hbm.at[pl.ds(base * NEXP, RPW * NEXP)], fl_v)

    lane = lax.broadcasted_iota(jnp.int32, (LANES,), 0)
    zeros_i = jnp.zeros((LANES,), jnp.int32)
    neg_inf = jnp.full((LANES,), -jnp.inf, jnp.float32)

    def group(g, _):
        rows = g * LANES + lane
        flat0 = rows * NEXP
        m1 = plsc.load_gather(fl_v, [flat0])
        i1 = zeros_i
        m2 = neg_inf
        i2 = zeros_i
        for e in range(1, NEXP):
            v = plsc.load_gather(fl_v, [flat0 + e])
            gt1 = v > m1
            gt2 = v > m2
            i2 = jnp.where(gt1, i1, jnp.where(gt2, e, i2))
            m2 = jnp.where(gt1, m1, jnp.where(gt2, v, m2))
            i1 = jnp.where(gt1, e, i1)
            m1 = jnp.where(gt1, v, m1)
        s = m1 + m2
        p1 = m1 / s
        p2 = m2 / s
        for e in range(NEXP):
            val = jnp.where(i1 == e, p1, jnp.where(i2 == e, p2, 0.0))
            plsc.store_scatter(sp_v, [flat0 + e], val)
        plsc.store_scatter(ix_v, [rows * 2], i1)
        plsc.store_scatter(ix_v, [rows * 2 + 1], i2)
        return 0

    lax.fori_loop(0, RPW // LANES, group, 0)

    pltpu.sync_copy(sp_v, sparse_hbm.at[pl.ds(base * NEXP, RPW * NEXP)])
    pltpu.sync_copy(ix_v, ix_hbm.at[pl.ds(base * 2, RPW * 2)])


@jax.jit
def _router(h, W_w, b_w, W_n, b_n, noise):
    full = _dense_full(h, W_w, b_w, W_n, b_n, noise)
    sparse_flat, ix_flat = _sc_route(full.reshape(N * NEXP))
    return sparse_flat.reshape(N, NEXP), ix_flat.reshape(N, 2), full


def kernel(h, W_w, b_w, W_n, b_n, noise):
    return _router(h, W_w, b_w, W_n, b_n, noise)
